# baseline (speedup 1.0000x reference)
import functools

import jax
import jax.numpy as jnp
from jax import lax
from jax.experimental import pallas as pl
from jax.experimental.pallas import tpu as pltpu
from jax.experimental.pallas import tpu_sc as plsc

_NEG = 0.2
_NC = 2
_NS = 16
_LANES = 16
_BSZ = 40


def _dense1_body(x_ref, w_ref, as_ref, ad_ref, haug_ref, adrow_ref):
    h = jnp.dot(x_ref[...], w_ref[...], preferred_element_type=jnp.float32)
    asrc = jnp.dot(h, as_ref[...], preferred_element_type=jnp.float32)
    adst = jnp.dot(h, ad_ref[...], preferred_element_type=jnp.float32)
    r = h.shape[0]
    haug_ref[...] = jnp.concatenate(
        [h, asrc, jnp.ones((r, 8), jnp.float32)], axis=1)
    adrow_ref[...] = jnp.concatenate(
        [adst, jnp.zeros((r, 8), jnp.float32)], axis=1)


def _dense1(x, w1, a_s, a_d, r=1000):
    n, d = x.shape
    hf = w1.shape[1]
    nh = a_s.shape[1]
    return pl.pallas_call(
        _dense1_body,
        grid=(n // r,),
        in_specs=[
            pl.BlockSpec((r, d), lambda i: (i, 0)),
            pl.BlockSpec((d, hf), lambda i: (0, 0)),
            pl.BlockSpec((hf, nh), lambda i: (0, 0)),
            pl.BlockSpec((hf, nh), lambda i: (0, 0)),
        ],
        out_specs=[
            pl.BlockSpec((r, hf + 16), lambda i: (i, 0)),
            pl.BlockSpec((r, 16), lambda i: (i, 0)),
        ],
        out_shape=[
            jax.ShapeDtypeStruct((n, hf + 16), jnp.float32),
            jax.ShapeDtypeStruct((n, 16), jnp.float32),
        ],
    )(x, w1, a_s, a_d)


def _finish1_body(a0_ref, a1_ref, b1_ref, w2a_ref, w2d_ref, c40_ref,
                  haug2_ref, ad2_ref):
    acc = a0_ref[...] + a1_ref[...]
    parts = []
    for h in range(8):
        d = acc[:, 128 + h:129 + h] + 1e-16
        parts.append(acc[:, h * 16:(h + 1) * 16] / d)
    o = jnp.concatenate(parts, axis=1) + b1_ref[...]
    o = jnp.where(o > 0, o, jnp.exp(o) - 1.0)
    haug2_ref[...] = (
        jnp.dot(o, w2a_ref[...], preferred_element_type=jnp.float32)
        + c40_ref[...])
    ad2_ref[...] = jnp.dot(o, w2d_ref[...], preferred_element_type=jnp.float32)


def _finish1(acc0, acc1, b1row, w2a, w2d, c40, r=1000):
    n, wa = acc0.shape
    w2 = w2a.shape[1]
    return pl.pallas_call(
        _finish1_body,
        grid=(n // r,),
        in_specs=[
            pl.BlockSpec((r, wa), lambda i: (i, 0)),
            pl.BlockSpec((r, wa), lambda i: (i, 0)),
            pl.BlockSpec((1, 128), lambda i: (0, 0)),
            pl.BlockSpec((128, w2), lambda i: (0, 0)),
            pl.BlockSpec((128, 16), lambda i: (0, 0)),
            pl.BlockSpec((1, w2), lambda i: (0, 0)),
        ],
        out_specs=[
            pl.BlockSpec((r, w2), lambda i: (i, 0)),
            pl.BlockSpec((r, 16), lambda i: (i, 0)),
        ],
        out_shape=[
            jax.ShapeDtypeStruct((n, w2), jnp.float32),
            jax.ShapeDtypeStruct((n, 16), jnp.float32),
        ],
    )(acc0, acc1, b1row, w2a, w2d, c40)


def _finish2_body(a0_ref, a1_ref, b2_ref, out_ref):
    acc = a0_ref[...] + a1_ref[...]
    c = out_ref.shape[1]
    out_ref[...] = acc[:, :c] / (acc[:, c:c + 1] + 1e-16) + b2_ref[...]


def _finish2(acc0, acc1, b2row, r=1000):
    n, wa = acc0.shape
    c = b2row.shape[1]
    return pl.pallas_call(
        _finish2_body,
        grid=(n // r,),
        in_specs=[
            pl.BlockSpec((r, wa), lambda i: (i, 0)),
            pl.BlockSpec((r, wa), lambda i: (i, 0)),
            pl.BlockSpec((1, c), lambda i: (0, 0)),
        ],
        out_specs=pl.BlockSpec((r, c), lambda i: (i, 0)),
        out_shape=jax.ShapeDtypeStruct((n, c), jnp.float32),
    )(acc0, acc1, b2row)


def _make_edge_kernel(n, e_total, w, nheads):
    nw = _NC * _NS
    bsz = _BSZ
    epw = e_total // nw
    nchunks = epw // bsz
    niter = nchunks // 2
    n_pad = -(-n // 1280) * 1280
    rps = n_pad // _NS
    mesh = plsc.VectorSubcoreMesh(core_axis_name="c", subcore_axis_name="s")

    @functools.partial(
        pl.kernel,
        out_type=jax.ShapeDtypeStruct((_NC, n_pad, w), jnp.float32),
        mesh=mesh,
        compiler_params=pltpu.CompilerParams(use_tc_tiling_on_sc=False),
        scratch_types=[
            pltpu.VMEM((nchunks, bsz), jnp.int32),
            pltpu.VMEM((nchunks, bsz), jnp.int32),
            pltpu.VMEM((bsz, w), jnp.float32),
            pltpu.VMEM((bsz, w), jnp.float32),
            pltpu.VMEM((bsz, 16), jnp.float32),
            pltpu.VMEM((bsz, 16), jnp.float32),
            pltpu.VMEM_SHARED((n_pad, w), jnp.float32),
            pltpu.SemaphoreType.DMA,
            pltpu.SemaphoreType.DMA,
            pltpu.SemaphoreType.DMA,
            pltpu.SemaphoreType.DMA,
            pltpu.SemaphoreType.DMA,
            pltpu.SemaphoreType.DMA,
        ],
    )
    def edge_kernel(haug, adt, src3, dst3, accs,
                    idxs_v, idxd_v, rows0, rows1, adr0, adr1, acc_sh,
                    semr0, semr1, sema0, sema1, semw0, semw1):
        cid = lax.axis_index("c")
        sid = lax.axis_index("s")
        g = cid * _NS + sid

        def zero_body(i, carry):
            for k in range(w // _LANES):
                rows0[i, pl.ds(k * _LANES, _LANES)] = jnp.zeros(
                    (_LANES,), jnp.float32)
            return carry
        lax.fori_loop(0, bsz, zero_body, 0)
        for t in range(rps // bsz):
            pltpu.sync_copy(rows0, acc_sh.at[pl.ds(sid * rps + t * bsz,
                                                   bsz)])
        pltpu.sync_copy(src3.at[g], idxs_v)
        pltpu.sync_copy(dst3.at[g], idxd_v)
        plsc.subcore_barrier()

        def g_start(c, rows_b, adr_b, semr, sema):
            pltpu.async_copy(haug.at[idxs_v.at[c]], rows_b, semr)
            pltpu.async_copy(adt.at[idxd_v.at[c]], adr_b, sema)

        def g_wait(c, rows_b, adr_b, semr, sema):
            pltpu.make_async_copy(haug.at[idxs_v.at[c]], rows_b, semr).wait()
            pltpu.make_async_copy(adt.at[idxd_v.at[c]], adr_b, sema).wait()

        def s_start(c, rows_b, semw):
            pltpu.async_copy(rows_b, acc_sh.at[idxd_v.at[c]], semw, add=True)

        def s_wait(c, rows_b, semw):
            pltpu.make_async_copy(rows_b, acc_sh.at[idxd_v.at[c]],
                                  semw).wait()

        def compute(rows, adrows):
            def edge_body(i, ecarry):
                if nheads > 1:
                    av = rows[i, pl.ds(w - 16, 16)]
                    dv = adrows[i, pl.ds(0, 16)]
                    ev = av + dv
                    ev = jnp.where(ev > 0, ev, ev * _NEG)
                    exv = jnp.exp(ev)
                    rows[i, pl.ds(w - 16, 16)] = exv
                    for hh in range(nheads):
                        bc = lax.gather(
                            exv,
                            jnp.full((16, 1), hh, jnp.int32),
                            lax.GatherDimensionNumbers(
                                offset_dims=(),
                                collapsed_slice_dims=(0,),
                                start_index_map=(0,)),
                            (1,),
                            mode=lax.GatherScatterMode.PROMISE_IN_BOUNDS)
                        rows[i, pl.ds(hh * 16, 16)] = (
                            rows[i, pl.ds(hh * 16, 16)] * bc)
                else:
                    av = rows[i, pl.ds(w - 16, 16)]
                    dv = adrows[i, pl.ds(0, 16)]
                    s = av[9] + dv[0]
                    s = jnp.where(s > 0, s, s * _NEG)
                    exv = jnp.exp(lax.broadcast(s, (16,)))
                    for kk in range(w // _LANES):
                        rows[i, pl.ds(kk * _LANES, _LANES)] = (
                            rows[i, pl.ds(kk * _LANES, _LANES)] * exv)
                return ecarry
            lax.fori_loop(0, bsz, edge_body, 0, unroll=4)

        g_start(0, rows0, adr0, semr0, sema0)

        def iter_body(t, carry):
            c0 = 2 * t
            c1 = c0 + 1

            @pl.when(t > 0)
            def _():
                s_wait(c1 - 2, rows1, semw1)
            g_start(c1, rows1, adr1, semr1, sema1)
            g_wait(c0, rows0, adr0, semr0, sema0)
            compute(rows0, adr0)
            s_start(c0, rows0, semw0)
            g_wait(c1, rows1, adr1, semr1, sema1)
            compute(rows1, adr1)

            @pl.when(t < niter - 1)
            def _():
                s_wait(c0, rows0, semw0)
                g_start(c0 + 2, rows0, adr0, semr0, sema0)
            s_start(c1, rows1, semw1)
            return carry
        lax.fori_loop(0, niter, iter_body, 0)
        s_wait(nchunks - 2, rows0, semw0)
        s_wait(nchunks - 1, rows1, semw1)

        plsc.subcore_barrier()
        pltpu.sync_copy(acc_sh.at[pl.ds(sid * rps, rps)],
                        accs.at[cid].at[pl.ds(sid * rps, rps)])

    return edge_kernel


def kernel(x, edge_index, W1, att_src1, att_dst1, b1, W2, att_src2,
           att_dst2, b2):
    n, d = x.shape
    e = edge_index.shape[1]
    nh, f = att_src1.shape
    hf = nh * f
    c = W2.shape[1]

    rows_idx = jnp.arange(hf)
    a_s = jnp.zeros((hf, nh), jnp.float32).at[
        rows_idx, rows_idx // f].set(att_src1.reshape(-1))
    a_d = jnp.zeros((hf, nh), jnp.float32).at[
        rows_idx, rows_idx // f].set(att_dst1.reshape(-1))

    nw = _NC * _NS
    src = edge_index[0].reshape(nw, e // (nw * _BSZ), _BSZ)
    dst = edge_index[1].reshape(nw, e // (nw * _BSZ), _BSZ)

    haug, adt = _dense1(x, W1, a_s, a_d)
    accs1 = _make_edge_kernel(n, e, hf + 16, nh)(haug, adt, src, dst)

    w2a = jnp.concatenate([
        W2,
        jnp.zeros((hf, 1), jnp.float32),
        (W2 @ att_src2[0]).reshape(hf, 1),
        jnp.zeros((hf, 48 - c - 2), jnp.float32),
    ], axis=1)
    w2d = jnp.concatenate([
        (W2 @ att_dst2[0]).reshape(hf, 1),
        jnp.zeros((hf, 15), jnp.float32),
    ], axis=1)
    c40 = jnp.zeros((1, 48), jnp.float32).at[0, c].set(1.0)

    haug2, ad2 = _finish1(accs1[0, :n], accs1[1, :n], b1.reshape(1, hf),
                          w2a, w2d, c40)

    accs2 = _make_edge_kernel(n, e, 48, 1)(haug2, ad2, src, dst)
    return _finish2(accs2[0, :n], accs2[1, :n], b2.reshape(1, c))


# speedup vs baseline: 1.1474x; 1.1474x over previous
import functools

import jax
import jax.numpy as jnp
from jax import lax
from jax.experimental import pallas as pl
from jax.experimental.pallas import tpu as pltpu
from jax.experimental.pallas import tpu_sc as plsc

_NEG = 0.2
_NC = 2
_NS = 16
_LANES = 16
_BSZ = 80


def _dense1_body(x_ref, w_ref, as_ref, ad_ref, haug_ref, adrow_ref):
    h = jnp.dot(x_ref[...], w_ref[...], preferred_element_type=jnp.float32)
    asrc = jnp.dot(h, as_ref[...], preferred_element_type=jnp.float32)
    adst = jnp.dot(h, ad_ref[...], preferred_element_type=jnp.float32)
    r = h.shape[0]
    haug_ref[...] = jnp.concatenate(
        [h, asrc, jnp.ones((r, 8), jnp.float32)], axis=1)
    adrow_ref[...] = jnp.concatenate(
        [adst, jnp.zeros((r, 8), jnp.float32)], axis=1)


def _dense1(x, w1, a_s, a_d, r=1000):
    n, d = x.shape
    hf = w1.shape[1]
    nh = a_s.shape[1]
    return pl.pallas_call(
        _dense1_body,
        grid=(n // r,),
        in_specs=[
            pl.BlockSpec((r, d), lambda i: (i, 0)),
            pl.BlockSpec((d, hf), lambda i: (0, 0)),
            pl.BlockSpec((hf, nh), lambda i: (0, 0)),
            pl.BlockSpec((hf, nh), lambda i: (0, 0)),
        ],
        out_specs=[
            pl.BlockSpec((r, hf + 16), lambda i: (i, 0)),
            pl.BlockSpec((r, 16), lambda i: (i, 0)),
        ],
        out_shape=[
            jax.ShapeDtypeStruct((n, hf + 16), jnp.float32),
            jax.ShapeDtypeStruct((n, 16), jnp.float32),
        ],
    )(x, w1, a_s, a_d)


def _finish1_body(a0_ref, a1_ref, b1_ref, w2a_ref, w2d_ref, c40_ref,
                  haug2_ref, ad2_ref):
    acc = a0_ref[...] + a1_ref[...]
    parts = []
    for h in range(8):
        d = acc[:, 128 + h:129 + h] + 1e-16
        parts.append(acc[:, h * 16:(h + 1) * 16] / d)
    o = jnp.concatenate(parts, axis=1) + b1_ref[...]
    o = jnp.where(o > 0, o, jnp.exp(o) - 1.0)
    haug2_ref[...] = (
        jnp.dot(o, w2a_ref[...], preferred_element_type=jnp.float32)
        + c40_ref[...])
    ad2_ref[...] = jnp.dot(o, w2d_ref[...], preferred_element_type=jnp.float32)


def _finish1(acc0, acc1, b1row, w2a, w2d, c40, r=1000):
    n, wa = acc0.shape
    w2 = w2a.shape[1]
    return pl.pallas_call(
        _finish1_body,
        grid=(n // r,),
        in_specs=[
            pl.BlockSpec((r, wa), lambda i: (i, 0)),
            pl.BlockSpec((r, wa), lambda i: (i, 0)),
            pl.BlockSpec((1, 128), lambda i: (0, 0)),
            pl.BlockSpec((128, w2), lambda i: (0, 0)),
            pl.BlockSpec((128, 16), lambda i: (0, 0)),
            pl.BlockSpec((1, w2), lambda i: (0, 0)),
        ],
        out_specs=[
            pl.BlockSpec((r, w2), lambda i: (i, 0)),
            pl.BlockSpec((r, 16), lambda i: (i, 0)),
        ],
        out_shape=[
            jax.ShapeDtypeStruct((n, w2), jnp.float32),
            jax.ShapeDtypeStruct((n, 16), jnp.float32),
        ],
    )(acc0, acc1, b1row, w2a, w2d, c40)


def _finish2_body(a0_ref, a1_ref, b2_ref, out_ref):
    acc = a0_ref[...] + a1_ref[...]
    c = out_ref.shape[1]
    out_ref[...] = acc[:, :c] / (acc[:, c:c + 1] + 1e-16) + b2_ref[...]


def _finish2(acc0, acc1, b2row, r=1000):
    n, wa = acc0.shape
    c = b2row.shape[1]
    return pl.pallas_call(
        _finish2_body,
        grid=(n // r,),
        in_specs=[
            pl.BlockSpec((r, wa), lambda i: (i, 0)),
            pl.BlockSpec((r, wa), lambda i: (i, 0)),
            pl.BlockSpec((1, c), lambda i: (0, 0)),
        ],
        out_specs=pl.BlockSpec((r, c), lambda i: (i, 0)),
        out_shape=jax.ShapeDtypeStruct((n, c), jnp.float32),
    )(acc0, acc1, b2row)


def _make_edge_kernel(n, e_total, w, nheads):
    nw = _NC * _NS
    bsz = _BSZ
    epw = e_total // nw
    nchunks = epw // bsz
    assert nchunks % 2 == 1
    niter = nchunks // 2
    n_pad = -(-n // 1280) * 1280
    rps = n_pad // _NS
    mesh = plsc.VectorSubcoreMesh(core_axis_name="c", subcore_axis_name="s")

    @functools.partial(
        pl.kernel,
        out_type=jax.ShapeDtypeStruct((_NC, n_pad, w), jnp.float32),
        mesh=mesh,
        compiler_params=pltpu.CompilerParams(use_tc_tiling_on_sc=False),
        scratch_types=[
            pltpu.VMEM((nchunks, bsz), jnp.int32),
            pltpu.VMEM((bsz,), jnp.int32),
            pltpu.VMEM((bsz,), jnp.int32),
            pltpu.VMEM((bsz,), jnp.int32),
            pltpu.VMEM((bsz,), jnp.int32),
            pltpu.VMEM((bsz, w), jnp.float32),
            pltpu.VMEM((bsz, w), jnp.float32),
            pltpu.VMEM((bsz, 16), jnp.float32),
            pltpu.VMEM((bsz, 16), jnp.float32),
            pltpu.VMEM_SHARED((n_pad, w), jnp.float32),
            pltpu.SemaphoreType.DMA,
            pltpu.SemaphoreType.DMA,
            pltpu.SemaphoreType.DMA,
            pltpu.SemaphoreType.DMA,
            pltpu.SemaphoreType.DMA,
            pltpu.SemaphoreType.DMA,
        ],
    )
    def edge_kernel(haug, adt, idxp3, accs,
                    idxp_v, idxs0, idxd0, idxs1, idxd1,
                    rows0, rows1, adr0, adr1, acc_sh,
                    semr0, semr1, sema0, sema1, semw0, semw1):
        cid = lax.axis_index("c")
        sid = lax.axis_index("s")
        g = cid * _NS + sid

        def zero_body(i, carry):
            for k in range(w // _LANES):
                rows0[i, pl.ds(k * _LANES, _LANES)] = jnp.zeros(
                    (_LANES,), jnp.float32)
            return carry
        lax.fori_loop(0, bsz, zero_body, 0)
        for t in range(rps // bsz):
            pltpu.sync_copy(rows0, acc_sh.at[pl.ds(sid * rps + t * bsz,
                                                   bsz)])
        pltpu.sync_copy(idxp3.at[g], idxp_v)
        plsc.subcore_barrier()

        def g_start(c, idxs_b, idxd_b, rows_b, adr_b, semr, sema):
            for k in range(bsz // _LANES):
                v = idxp_v[c, pl.ds(k * _LANES, _LANES)]
                idxs_b[pl.ds(k * _LANES, _LANES)] = jnp.bitwise_and(
                    v, jnp.int32(0xFFFF))
                idxd_b[pl.ds(k * _LANES, _LANES)] = (
                    lax.shift_right_logical(v, jnp.int32(16)))
            pltpu.async_copy(haug.at[idxs_b], rows_b, semr)
            pltpu.async_copy(adt.at[idxd_b], adr_b, sema)

        def g_wait(idxs_b, idxd_b, rows_b, adr_b, semr, sema):
            pltpu.make_async_copy(haug.at[idxs_b], rows_b, semr).wait()
            pltpu.make_async_copy(adt.at[idxd_b], adr_b, sema).wait()

        def s_start(idxd_b, rows_b, semw):
            pltpu.async_copy(rows_b, acc_sh.at[idxd_b], semw, add=True)

        def s_wait(idxd_b, rows_b, semw):
            pltpu.make_async_copy(rows_b, acc_sh.at[idxd_b], semw).wait()

        def compute(rows, adrows):
            def edge_body(i, ecarry):
                if nheads > 1:
                    av = rows[i, pl.ds(w - 16, 16)]
                    dv = adrows[i, pl.ds(0, 16)]
                    ev = av + dv
                    ev = jnp.where(ev > 0, ev, ev * _NEG)
                    exv = jnp.exp(ev)
                    rows[i, pl.ds(w - 16, 16)] = exv
                    for hh in range(nheads):
                        bc = lax.gather(
                            exv,
                            jnp.full((16, 1), hh, jnp.int32),
                            lax.GatherDimensionNumbers(
                                offset_dims=(),
                                collapsed_slice_dims=(0,),
                                start_index_map=(0,)),
                            (1,),
                            mode=lax.GatherScatterMode.PROMISE_IN_BOUNDS)
                        rows[i, pl.ds(hh * 16, 16)] = (
                            rows[i, pl.ds(hh * 16, 16)] * bc)
                else:
                    av = rows[i, pl.ds(w - 16, 16)]
                    dv = adrows[i, pl.ds(0, 16)]
                    s = av[9] + dv[0]
                    s = jnp.where(s > 0, s, s * _NEG)
                    exv = jnp.exp(lax.broadcast(s, (16,)))
                    for kk in range(w // _LANES):
                        rows[i, pl.ds(kk * _LANES, _LANES)] = (
                            rows[i, pl.ds(kk * _LANES, _LANES)] * exv)
                return ecarry
            lax.fori_loop(0, bsz, edge_body, 0, unroll=4)

        g_start(0, idxs0, idxd0, rows0, adr0, semr0, sema0)

        def iter_body(t, carry):
            c0 = 2 * t
            c1 = c0 + 1

            @pl.when(t > 0)
            def _():
                s_wait(idxd1, rows1, semw1)
            g_start(c1, idxs1, idxd1, rows1, adr1, semr1, sema1)
            g_wait(idxs0, idxd0, rows0, adr0, semr0, sema0)
            compute(rows0, adr0)
            s_start(idxd0, rows0, semw0)
            s_wait(idxd0, rows0, semw0)
            g_start(c0 + 2, idxs0, idxd0, rows0, adr0, semr0, sema0)
            g_wait(idxs1, idxd1, rows1, adr1, semr1, sema1)
            compute(rows1, adr1)
            s_start(idxd1, rows1, semw1)
            return carry
        lax.fori_loop(0, niter, iter_body, 0)
        s_wait(idxd1, rows1, semw1)
        g_wait(idxs0, idxd0, rows0, adr0, semr0, sema0)
        compute(rows0, adr0)
        s_start(idxd0, rows0, semw0)
        s_wait(idxd0, rows0, semw0)

        plsc.subcore_barrier()
        pltpu.sync_copy(acc_sh.at[pl.ds(sid * rps, rps)],
                        accs.at[cid].at[pl.ds(sid * rps, rps)])

    return edge_kernel


def kernel(x, edge_index, W1, att_src1, att_dst1, b1, W2, att_src2,
           att_dst2, b2):
    n, d = x.shape
    e = edge_index.shape[1]
    nh, f = att_src1.shape
    hf = nh * f
    c = W2.shape[1]

    rows_idx = jnp.arange(hf)
    a_s = jnp.zeros((hf, nh), jnp.float32).at[
        rows_idx, rows_idx // f].set(att_src1.reshape(-1))
    a_d = jnp.zeros((hf, nh), jnp.float32).at[
        rows_idx, rows_idx // f].set(att_dst1.reshape(-1))

    nw = _NC * _NS
    idxp = jnp.bitwise_or(
        jnp.left_shift(edge_index[1], jnp.int32(16)), edge_index[0]
    ).reshape(nw, e // (nw * _BSZ), _BSZ)

    haug, adt = _dense1(x, W1, a_s, a_d)
    accs1 = _make_edge_kernel(n, e, hf + 16, nh)(haug, adt, idxp)

    w2a = jnp.concatenate([
        W2,
        jnp.zeros((hf, 1), jnp.float32),
        (W2 @ att_src2[0]).reshape(hf, 1),
        jnp.zeros((hf, 48 - c - 2), jnp.float32),
    ], axis=1)
    w2d = jnp.concatenate([
        (W2 @ att_dst2[0]).reshape(hf, 1),
        jnp.zeros((hf, 15), jnp.float32),
    ], axis=1)
    c40 = jnp.zeros((1, 48), jnp.float32).at[0, c].set(1.0)

    haug2, ad2 = _finish1(accs1[0, :n], accs1[1, :n], b1.reshape(1, hf),
                          w2a, w2d, c40)

    accs2 = _make_edge_kernel(n, e, 48, 1)(haug2, ad2, idxp)
    return _finish2(accs2[0, :n], accs2[1, :n], b2.reshape(1, c))


# speedup vs baseline: 2.2884x; 1.9944x over previous
import functools

import jax
import jax.numpy as jnp
from jax import lax
from jax.experimental import pallas as pl
from jax.experimental.pallas import tpu as pltpu
from jax.experimental.pallas import tpu_sc as plsc

_NEG = 0.2
_NC = 2
_NS = 16
_LANES = 16
_BSZ = 80


def _dense1_body(x_ref, w_ref, as_ref, ad_ref, haug_ref, adrow_ref):
    h = jnp.dot(x_ref[...], w_ref[...], preferred_element_type=jnp.float32)
    asrc = jnp.dot(h, as_ref[...], preferred_element_type=jnp.float32)
    adst = jnp.dot(h, ad_ref[...], preferred_element_type=jnp.float32)
    r = h.shape[0]
    haug_ref[...] = jnp.concatenate(
        [h, asrc, jnp.ones((r, 8), jnp.float32)], axis=1)
    adrow_ref[...] = jnp.concatenate(
        [adst, jnp.zeros((r, 8), jnp.float32)], axis=1)


def _dense1(x, w1, a_s, a_d, r=1000):
    n, d = x.shape
    hf = w1.shape[1]
    nh = a_s.shape[1]
    return pl.pallas_call(
        _dense1_body,
        grid=(n // r,),
        in_specs=[
            pl.BlockSpec((r, d), lambda i: (i, 0)),
            pl.BlockSpec((d, hf), lambda i: (0, 0)),
            pl.BlockSpec((hf, nh), lambda i: (0, 0)),
            pl.BlockSpec((hf, nh), lambda i: (0, 0)),
        ],
        out_specs=[
            pl.BlockSpec((r, hf + 16), lambda i: (i, 0)),
            pl.BlockSpec((r, 16), lambda i: (i, 0)),
        ],
        out_shape=[
            jax.ShapeDtypeStruct((n, hf + 16), jnp.float32),
            jax.ShapeDtypeStruct((n, 16), jnp.float32),
        ],
    )(x, w1, a_s, a_d)


def _finish1_body(a0_ref, a1_ref, b1_ref, w2a_ref, w2d_ref, c40_ref,
                  haug2_ref, ad2_ref):
    acc = a0_ref[...] + a1_ref[...]
    parts = []
    for h in range(8):
        d = acc[:, 128 + h:129 + h] + 1e-16
        parts.append(acc[:, h * 16:(h + 1) * 16] / d)
    o = jnp.concatenate(parts, axis=1) + b1_ref[...]
    o = jnp.where(o > 0, o, jnp.exp(o) - 1.0)
    haug2_ref[...] = (
        jnp.dot(o, w2a_ref[...], preferred_element_type=jnp.float32)
        + c40_ref[...])
    ad2_ref[...] = jnp.dot(o, w2d_ref[...], preferred_element_type=jnp.float32)


def _finish1(acc0, acc1, b1row, w2a, w2d, c40, r=1000):
    n, wa = acc0.shape
    w2 = w2a.shape[1]
    return pl.pallas_call(
        _finish1_body,
        grid=(n // r,),
        in_specs=[
            pl.BlockSpec((r, wa), lambda i: (i, 0)),
            pl.BlockSpec((r, wa), lambda i: (i, 0)),
            pl.BlockSpec((1, 128), lambda i: (0, 0)),
            pl.BlockSpec((128, w2), lambda i: (0, 0)),
            pl.BlockSpec((128, 16), lambda i: (0, 0)),
            pl.BlockSpec((1, w2), lambda i: (0, 0)),
        ],
        out_specs=[
            pl.BlockSpec((r, w2), lambda i: (i, 0)),
            pl.BlockSpec((r, 16), lambda i: (i, 0)),
        ],
        out_shape=[
            jax.ShapeDtypeStruct((n, w2), jnp.float32),
            jax.ShapeDtypeStruct((n, 16), jnp.float32),
        ],
    )(acc0, acc1, b1row, w2a, w2d, c40)


def _finish2_body(a0_ref, a1_ref, b2_ref, out_ref):
    acc = a0_ref[...] + a1_ref[...]
    c = out_ref.shape[1]
    out_ref[...] = acc[:, :c] / (acc[:, c:c + 1] + 1e-16) + b2_ref[...]


def _finish2(acc0, acc1, b2row, r=1000):
    n, wa = acc0.shape
    c = b2row.shape[1]
    return pl.pallas_call(
        _finish2_body,
        grid=(n // r,),
        in_specs=[
            pl.BlockSpec((r, wa), lambda i: (i, 0)),
            pl.BlockSpec((r, wa), lambda i: (i, 0)),
            pl.BlockSpec((1, c), lambda i: (0, 0)),
        ],
        out_specs=pl.BlockSpec((r, c), lambda i: (i, 0)),
        out_shape=jax.ShapeDtypeStruct((n, c), jnp.float32),
    )(acc0, acc1, b2row)


def _make_edge_kernel(n, e_total, w, nheads):
    nw = _NC * _NS
    bsz = _BSZ
    epw = e_total // nw
    nchunks = epw // bsz
    assert nchunks % 2 == 1
    niter = nchunks // 2
    n_pad = -(-n // 1280) * 1280
    rps = n_pad // _NS
    mesh = plsc.VectorSubcoreMesh(core_axis_name="c", subcore_axis_name="s")

    @functools.partial(
        pl.kernel,
        out_type=jax.ShapeDtypeStruct((_NC, n_pad, w), jnp.float32),
        mesh=mesh,
        compiler_params=pltpu.CompilerParams(use_tc_tiling_on_sc=False),
        scratch_types=[
            pltpu.VMEM((nchunks, bsz), jnp.int32),
            pltpu.VMEM((bsz,), jnp.int32),
            pltpu.VMEM((bsz,), jnp.int32),
            pltpu.VMEM((bsz,), jnp.int32),
            pltpu.VMEM((bsz,), jnp.int32),
            pltpu.VMEM((bsz, w), jnp.float32),
            pltpu.VMEM((bsz, w), jnp.float32),
            pltpu.VMEM((bsz, 16), jnp.float32),
            pltpu.VMEM((bsz, 16), jnp.float32),
            pltpu.VMEM_SHARED((n_pad, w), jnp.float32),
            pltpu.SemaphoreType.DMA,
            pltpu.SemaphoreType.DMA,
            pltpu.SemaphoreType.DMA,
            pltpu.SemaphoreType.DMA,
            pltpu.SemaphoreType.DMA,
            pltpu.SemaphoreType.DMA,
        ],
    )
    def edge_kernel(haug, adt, idxp3, accs,
                    idxp_v, idxs0, idxd0, idxs1, idxd1,
                    rows0, rows1, adr0, adr1, acc_sh,
                    semr0, semr1, sema0, sema1, semw0, semw1):
        cid = lax.axis_index("c")
        sid = lax.axis_index("s")
        g = cid * _NS + sid

        def zero_body(i, carry):
            for k in range(w // _LANES):
                rows0[i, pl.ds(k * _LANES, _LANES)] = jnp.zeros(
                    (_LANES,), jnp.float32)
            return carry
        lax.fori_loop(0, bsz, zero_body, 0)
        for t in range(rps // bsz):
            pltpu.sync_copy(rows0, acc_sh.at[pl.ds(sid * rps + t * bsz,
                                                   bsz)])
        pltpu.sync_copy(idxp3.at[g], idxp_v)
        plsc.subcore_barrier()

        def g_start(c, idxs_b, idxd_b, rows_b, adr_b, semr, sema):
            for k in range(bsz // _LANES):
                v = idxp_v[c, pl.ds(k * _LANES, _LANES)]
                idxs_b[pl.ds(k * _LANES, _LANES)] = jnp.bitwise_and(
                    v, jnp.int32(0xFFFF))
                idxd_b[pl.ds(k * _LANES, _LANES)] = (
                    lax.shift_right_logical(v, jnp.int32(16)))
            pltpu.async_copy(haug.at[idxs_b], rows_b, semr)
            pltpu.async_copy(adt.at[idxd_b], adr_b, sema)

        def g_wait(idxs_b, idxd_b, rows_b, adr_b, semr, sema):
            pltpu.make_async_copy(haug.at[idxs_b], rows_b, semr).wait()
            pltpu.make_async_copy(adt.at[idxd_b], adr_b, sema).wait()

        def s_start(idxd_b, rows_b, semw):
            pltpu.async_copy(rows_b, acc_sh.at[idxd_b], semw, add=True)

        def s_wait(idxd_b, rows_b, semw):
            pltpu.make_async_copy(rows_b, acc_sh.at[idxd_b], semw).wait()

        def compute(rows, adrows):
            @functools.partial(plsc.parallel_loop, 0, bsz, unroll=4)
            def edge_body(i):
                if nheads > 1:
                    av = rows[i, pl.ds(w - 16, 16)]
                    dv = adrows[i, pl.ds(0, 16)]
                    ev = av + dv
                    ev = jnp.where(ev > 0, ev, ev * _NEG)
                    exv = jnp.exp(ev)
                    rows[i, pl.ds(w - 16, 16)] = exv
                    for hh in range(nheads):
                        bc = lax.gather(
                            exv,
                            jnp.full((16, 1), hh, jnp.int32),
                            lax.GatherDimensionNumbers(
                                offset_dims=(),
                                collapsed_slice_dims=(0,),
                                start_index_map=(0,)),
                            (1,),
                            mode=lax.GatherScatterMode.PROMISE_IN_BOUNDS)
                        rows[i, pl.ds(hh * 16, 16)] = (
                            rows[i, pl.ds(hh * 16, 16)] * bc)
                else:
                    av = rows[i, pl.ds(w - 16, 16)]
                    dv = adrows[i, pl.ds(0, 16)]
                    s = av[9] + dv[0]
                    s = jnp.where(s > 0, s, s * _NEG)
                    exv = jnp.exp(lax.broadcast(s, (16,)))
                    for kk in range(w // _LANES):
                        rows[i, pl.ds(kk * _LANES, _LANES)] = (
                            rows[i, pl.ds(kk * _LANES, _LANES)] * exv)

        g_start(0, idxs0, idxd0, rows0, adr0, semr0, sema0)

        def iter_body(t, carry):
            c0 = 2 * t
            c1 = c0 + 1

            @pl.when(t > 0)
            def _():
                s_wait(idxd1, rows1, semw1)
            g_start(c1, idxs1, idxd1, rows1, adr1, semr1, sema1)
            g_wait(idxs0, idxd0, rows0, adr0, semr0, sema0)
            compute(rows0, adr0)
            s_start(idxd0, rows0, semw0)
            s_wait(idxd0, rows0, semw0)
            g_start(c0 + 2, idxs0, idxd0, rows0, adr0, semr0, sema0)
            g_wait(idxs1, idxd1, rows1, adr1, semr1, sema1)
            compute(rows1, adr1)
            s_start(idxd1, rows1, semw1)
            return carry
        lax.fori_loop(0, niter, iter_body, 0)
        s_wait(idxd1, rows1, semw1)
        g_wait(idxs0, idxd0, rows0, adr0, semr0, sema0)
        compute(rows0, adr0)
        s_start(idxd0, rows0, semw0)
        s_wait(idxd0, rows0, semw0)

        plsc.subcore_barrier()
        pltpu.sync_copy(acc_sh.at[pl.ds(sid * rps, rps)],
                        accs.at[cid].at[pl.ds(sid * rps, rps)])

    return edge_kernel


def kernel(x, edge_index, W1, att_src1, att_dst1, b1, W2, att_src2,
           att_dst2, b2):
    n, d = x.shape
    e = edge_index.shape[1]
    nh, f = att_src1.shape
    hf = nh * f
    c = W2.shape[1]

    rows_idx = jnp.arange(hf)
    a_s = jnp.zeros((hf, nh), jnp.float32).at[
        rows_idx, rows_idx // f].set(att_src1.reshape(-1))
    a_d = jnp.zeros((hf, nh), jnp.float32).at[
        rows_idx, rows_idx // f].set(att_dst1.reshape(-1))

    nw = _NC * _NS
    idxp = jnp.bitwise_or(
        jnp.left_shift(edge_index[1], jnp.int32(16)), edge_index[0]
    ).reshape(nw, e // (nw * _BSZ), _BSZ)

    haug, adt = _dense1(x, W1, a_s, a_d)
    accs1 = _make_edge_kernel(n, e, hf + 16, nh)(haug, adt, idxp)

    w2a = jnp.concatenate([
        W2,
        jnp.zeros((hf, 1), jnp.float32),
        (W2 @ att_src2[0]).reshape(hf, 1),
        jnp.zeros((hf, 48 - c - 2), jnp.float32),
    ], axis=1)
    w2d = jnp.concatenate([
        (W2 @ att_dst2[0]).reshape(hf, 1),
        jnp.zeros((hf, 15), jnp.float32),
    ], axis=1)
    c40 = jnp.zeros((1, 48), jnp.float32).at[0, c].set(1.0)

    haug2, ad2 = _finish1(accs1[0, :n], accs1[1, :n], b1.reshape(1, hf),
                          w2a, w2d, c40)

    accs2 = _make_edge_kernel(n, e, 48, 1)(haug2, ad2, idxp)
    return _finish2(accs2[0, :n], accs2[1, :n], b2.reshape(1, c))
